# baseline (speedup 1.0000x reference)
import functools

import jax
import jax.numpy as jnp
from jax import lax
from jax.experimental import pallas as pl
from jax.experimental.pallas import tpu as pltpu
from jax.experimental.pallas import tpu_sc as plsc

N = 50000
E = 800000
F = 64
FH = 32
OUT = 6

NP = 50176
SL = NP // 16
CH = 128
NCHUNK = 6272
EP = NCHUNK * CH
WCHUNK = NCHUNK // 32
TCHUNK = NCHUNK // 16
BCH = 28
RBT = 1792
NBLK = NP // RBT

_MESH = plsc.VectorSubcoreMesh(core_axis_name="c", subcore_axis_name="s")
_CP = pltpu.CompilerParams(
    needs_layout_passes=False, use_tc_tiling_on_sc=False)


def _sc_degree(dstp):
    @functools.partial(
        pl.kernel,
        out_type=jax.ShapeDtypeStruct((32, NP), jnp.float32),
        mesh=_MESH,
        compiler_params=_CP,
        scratch_types=[
            pltpu.VMEM((NP,), jnp.float32),
            pltpu.VMEM((WCHUNK * CH,), jnp.int32),
        ],
    )
    def k(dst_hbm, deg_hbm, hist, ibuf):
        cid = lax.axis_index("c")
        sid = lax.axis_index("s")
        wid = cid * 16 + sid
        zeros = jnp.zeros((16,), jnp.float32)
        ones = jnp.ones((16,), jnp.float32)

        @pl.loop(0, NP, step=16)
        def _(i):
            hist[pl.ds(i, 16)] = zeros

        pltpu.sync_copy(dst_hbm.at[pl.ds(wid * WCHUNK * CH, WCHUNK * CH)], ibuf)

        @pl.loop(0, WCHUNK * CH, step=16)
        def _(i):
            plsc.addupdate_scatter(hist, [ibuf[pl.ds(i, 16)]], ones)

        pltpu.sync_copy(hist, deg_hbm.at[wid])

    return k(dstp)


def _sc_aggregate(gl, gr, srcp, dstp, zslab):
    @functools.partial(
        pl.kernel,
        out_type=jax.ShapeDtypeStruct((2, NP, FH), jnp.float32),
        mesh=_MESH,
        compiler_params=_CP,
        scratch_types=[
            pltpu.VMEM((BCH, CH), jnp.int32),
            pltpu.VMEM((BCH, CH), jnp.int32),
            pltpu.VMEM((4, CH, FH), jnp.float32),
            pltpu.VMEM((WCHUNK // 2, FH), jnp.float32),
            pltpu.VMEM_SHARED((NP, FH), jnp.float32),
            pltpu.SemaphoreType.DMA,
            pltpu.SemaphoreType.DMA,
        ],
    )
    def k(gl_hbm, gr_hbm, src_hbm, dst_hbm, z_hbm, agg_hbm,
          sbuf, dbuf, rows, wb, acc, gsem, ssem):
        cid = lax.axis_index("c")
        sid = lax.axis_index("s")
        row0 = sid * SL

        pltpu.sync_copy(z_hbm, acc.at[pl.ds(row0, SL)])
        plsc.subcore_barrier()

        def edge_pass(g_hbm):
            @pl.loop(0, TCHUNK // BCH)
            def _(b):
                ch0 = sid * TCHUNK + b * BCH
                pltpu.sync_copy(src_hbm.at[pl.ds(ch0, BCH)], sbuf)
                pltpu.sync_copy(dst_hbm.at[pl.ds(ch0, BCH)], dbuf)

                @pl.loop(0, BCH, step=4)
                def _(j):
                    gds = [
                        pltpu.async_copy(
                            g_hbm.at[sbuf.at[j + q]], rows.at[q], gsem)
                        for q in range(2)
                    ]
                    sds = []
                    for q in range(2):
                        gds[q].wait()
                        sds.append(pltpu.async_copy(
                            rows.at[q], acc.at[dbuf.at[j + q]], ssem,
                            add=True))
                    gds2 = [
                        pltpu.async_copy(
                            g_hbm.at[sbuf.at[j + 2 + q]], rows.at[2 + q], gsem)
                        for q in range(2)
                    ]
                    for q in range(2):
                        gds2[q].wait()
                        sds.append(pltpu.async_copy(
                            rows.at[2 + q], acc.at[dbuf.at[j + 2 + q]], ssem,
                            add=True))
                    for d in sds:
                        d.wait()

        @pl.when(cid == 0)
        def _():
            edge_pass(gl_hbm)

        @pl.when(cid == 1)
        def _():
            edge_pass(gr_hbm)

        plsc.subcore_barrier()

        @pl.loop(0, 32)
        def _(kk):
            half = WCHUNK // 2
            pltpu.sync_copy(acc.at[pl.ds(row0 + kk * half, half)], wb)
            pltpu.sync_copy(wb, agg_hbm.at[cid, pl.ds(row0 + kk * half, half)])

    return k(gl, gr, srcp.reshape(NCHUNK, CH), dstp.reshape(NCHUNK, CH), zslab)


def _sc_t(srcp, dstp, dinv):
    @functools.partial(
        pl.kernel,
        out_type=jax.ShapeDtypeStruct((32, NP), jnp.float32),
        mesh=_MESH,
        compiler_params=_CP,
        scratch_types=[
            pltpu.VMEM((NP,), jnp.float32),
            pltpu.VMEM((NP,), jnp.float32),
            pltpu.VMEM((BCH * CH,), jnp.int32),
            pltpu.VMEM((BCH * CH,), jnp.int32),
        ],
    )
    def k(src_hbm, dst_hbm, dinv_hbm, t_hbm, dloc, th, sbuf, dbuf):
        cid = lax.axis_index("c")
        sid = lax.axis_index("s")
        wid = cid * 16 + sid
        zeros = jnp.zeros((16,), jnp.float32)

        pltpu.sync_copy(dinv_hbm, dloc)

        @pl.loop(0, NP, step=16)
        def _(i):
            th[pl.ds(i, 16)] = zeros

        @pl.loop(0, WCHUNK // BCH)
        def _(b):
            e0 = (wid * WCHUNK + b * BCH) * CH
            pltpu.sync_copy(src_hbm.at[pl.ds(e0, BCH * CH)], sbuf)
            pltpu.sync_copy(dst_hbm.at[pl.ds(e0, BCH * CH)], dbuf)

            @pl.loop(0, BCH * CH, step=16)
            def _(i):
                dvals = plsc.load_gather(dloc, [dbuf[pl.ds(i, 16)]])
                plsc.addupdate_scatter(th, [sbuf[pl.ds(i, 16)]], dvals)

        pltpu.sync_copy(th, t_hbm.at[wid])

    return k(srcp, dstp, dinv)


def _tc_matmul(xp, W1):
    def body(x_ref, w_ref, h_ref):
        h_ref[...] = lax.dot_general(
            x_ref[...], w_ref[...], (((1,), (1,)), ((), ())),
            preferred_element_type=jnp.float32)

    return pl.pallas_call(
        body,
        grid=(NBLK,),
        in_specs=[
            pl.BlockSpec((RBT, F), lambda i: (i, 0)),
            pl.BlockSpec((F, F), lambda i: (0, 0)),
        ],
        out_specs=pl.BlockSpec((RBT, F), lambda i: (i, 0)),
        out_shape=jax.ShapeDtypeStruct((NP, F), jnp.float32),
    )(xp, W1)


def _tc_scale(deg32, h):
    def body(deg_ref, h_ref, dinv_ref, dinv1_ref, gl_ref, gr_ref):
        i = pl.program_id(0)
        deg = jnp.sum(deg_ref[...], axis=0) + 1.0
        dv = lax.rsqrt(deg)
        dvc = dv[:, None]
        dinv_ref[...] = dvc
        dinv1_ref[pl.ds(i * RBT, RBT)] = dv
        g = h_ref[...] * dvc
        gl_ref[...] = g[:, :FH]
        gr_ref[...] = g[:, FH:]

    return pl.pallas_call(
        body,
        grid=(NBLK,),
        in_specs=[
            pl.BlockSpec((32, RBT), lambda i: (0, i)),
            pl.BlockSpec((RBT, F), lambda i: (i, 0)),
        ],
        out_specs=[
            pl.BlockSpec((RBT, 1), lambda i: (i, 0)),
            pl.BlockSpec((NP,), lambda i: (0,)),
            pl.BlockSpec((RBT, FH), lambda i: (i, 0)),
            pl.BlockSpec((RBT, FH), lambda i: (i, 0)),
        ],
        out_shape=[
            jax.ShapeDtypeStruct((NP, 1), jnp.float32),
            jax.ShapeDtypeStruct((NP,), jnp.float32),
            jax.ShapeDtypeStruct((NP, FH), jnp.float32),
            jax.ShapeDtypeStruct((NP, FH), jnp.float32),
        ],
    )(deg32, h)


def _tc_final(agg2, gl, gr, dinv, t32, b1, W2T, b2):
    def body(agg_ref, gl_ref, gr_ref, dinv_ref, t_ref, b1_ref,
             w2t_ref, b2_ref, res_ref, acc):
        i = pl.program_id(0)
        dvc = dinv_ref[...]
        pre_l = (agg_ref[0] + gl_ref[...]) * dvc
        pre_r = (agg_ref[1] + gr_ref[...]) * dvc
        pre = jnp.concatenate([pre_l, pre_r], axis=1) + b1_ref[...]
        r1 = jnp.maximum(pre, 0.0)
        tsum = jnp.sum(t_ref[...], axis=0)[:, None]
        w = dvc * (tsum + dvc)
        rowid = lax.broadcasted_iota(jnp.int32, (RBT, 1), 0) + i * RBT
        w = jnp.where(rowid < N, w, 0.0)
        part = jnp.sum(r1 * w, axis=0, keepdims=True)

        @pl.when(i == 0)
        def _():
            acc[...] = part

        @pl.when(i > 0)
        def _():
            acc[...] += part

        @pl.when(i == NBLK - 1)
        def _():
            r = acc[...] * (1.0 / N)
            res_ref[...] = jnp.dot(
                r, w2t_ref[...], preferred_element_type=jnp.float32) + b2_ref[...]

    return pl.pallas_call(
        body,
        grid=(NBLK,),
        in_specs=[
            pl.BlockSpec((2, RBT, FH), lambda i: (0, i, 0)),
            pl.BlockSpec((RBT, FH), lambda i: (i, 0)),
            pl.BlockSpec((RBT, FH), lambda i: (i, 0)),
            pl.BlockSpec((RBT, 1), lambda i: (i, 0)),
            pl.BlockSpec((32, RBT), lambda i: (0, i)),
            pl.BlockSpec((1, F), lambda i: (0, 0)),
            pl.BlockSpec((F, OUT), lambda i: (0, 0)),
            pl.BlockSpec((1, OUT), lambda i: (0, 0)),
        ],
        out_specs=pl.BlockSpec((1, OUT), lambda i: (0, 0)),
        out_shape=jax.ShapeDtypeStruct((1, OUT), jnp.float32),
        scratch_shapes=[pltpu.VMEM((1, F), jnp.float32)],
    )(agg2, gl, gr, dinv, t32, b1, W2T, b2)


def kernel(x, edge_index, W1, b1, W2, b2):
    x = x.astype(jnp.float32)
    xp = jnp.concatenate(
        [x, jnp.zeros((NP - N, F), jnp.float32)], axis=0)

    src = edge_index[0].astype(jnp.int32)
    dst = edge_index[1].astype(jnp.int32)
    padlen = EP - E
    pad_vals = N + (jnp.arange(padlen, dtype=jnp.int32) % (NP - N))
    srcp = jnp.concatenate([src, pad_vals])
    dstp = jnp.concatenate([dst, pad_vals])

    zslab = jnp.zeros((SL, FH), jnp.float32)

    deg32 = _sc_degree(dstp)
    h = _tc_matmul(xp, W1.astype(jnp.float32))
    dinv, dinv1, gl, gr = _tc_scale(deg32, h)
    agg2 = _sc_aggregate(gl, gr, srcp, dstp, zslab)
    t32 = _sc_t(srcp, dstp, dinv1)
    res = _tc_final(
        agg2, gl, gr, dinv, t32,
        b1.astype(jnp.float32).reshape(1, F),
        W2.astype(jnp.float32).T,
        b2.astype(jnp.float32).reshape(1, OUT),
    )
    return res.reshape(OUT)


# speedup vs baseline: 1.1123x; 1.1123x over previous
import functools

import jax
import jax.numpy as jnp
from jax import lax
from jax.experimental import pallas as pl
from jax.experimental.pallas import tpu as pltpu
from jax.experimental.pallas import tpu_sc as plsc

N = 50000
E = 800000
F = 64
FH = 32
OUT = 6

NP = 50176
SL = NP // 16
CH = 128
NCHUNK = 6272
EP = NCHUNK * CH
WCHUNK = NCHUNK // 32
TCHUNK = NCHUNK // 16
BCH = 28
RBT = 1792
NBLK = NP // RBT

_MESH = plsc.VectorSubcoreMesh(core_axis_name="c", subcore_axis_name="s")
_CP = pltpu.CompilerParams(
    needs_layout_passes=False, use_tc_tiling_on_sc=False)


def _sc_degree(dstp):
    @functools.partial(
        pl.kernel,
        out_type=jax.ShapeDtypeStruct((32, NP), jnp.float32),
        mesh=_MESH,
        compiler_params=_CP,
        scratch_types=[
            pltpu.VMEM((NP,), jnp.float32),
            pltpu.VMEM((WCHUNK * CH,), jnp.int32),
        ],
    )
    def k(dst_hbm, deg_hbm, hist, ibuf):
        cid = lax.axis_index("c")
        sid = lax.axis_index("s")
        wid = cid * 16 + sid
        zeros = jnp.zeros((16,), jnp.float32)
        ones = jnp.ones((16,), jnp.float32)

        @pl.loop(0, NP, step=16)
        def _(i):
            hist[pl.ds(i, 16)] = zeros

        pltpu.sync_copy(dst_hbm.at[pl.ds(wid * WCHUNK * CH, WCHUNK * CH)], ibuf)

        @pl.loop(0, WCHUNK * CH, step=16)
        def _(i):
            plsc.addupdate_scatter(hist, [ibuf[pl.ds(i, 16)]], ones)

        pltpu.sync_copy(hist, deg_hbm.at[wid])

    return k(dstp)


def _sc_aggregate(gl, gr, srcp, dstp, zslab):
    @functools.partial(
        pl.kernel,
        out_type=jax.ShapeDtypeStruct((2, NP, FH), jnp.float32),
        mesh=_MESH,
        compiler_params=_CP,
        scratch_types=[
            pltpu.VMEM((BCH, CH), jnp.int32),
            pltpu.VMEM((BCH, CH), jnp.int32),
            pltpu.VMEM((4, CH, FH), jnp.float32),
            pltpu.VMEM((WCHUNK // 2, FH), jnp.float32),
            pltpu.VMEM_SHARED((NP, FH), jnp.float32),
            pltpu.SemaphoreType.DMA,
            pltpu.SemaphoreType.DMA,
        ],
    )
    def k(gl_hbm, gr_hbm, src_hbm, dst_hbm, z_hbm, agg_hbm,
          sbuf, dbuf, rows, wb, acc, gsem, ssem):
        cid = lax.axis_index("c")
        sid = lax.axis_index("s")
        row0 = sid * SL

        pltpu.sync_copy(z_hbm, acc.at[pl.ds(row0, SL)])
        plsc.subcore_barrier()

        def edge_pass(g_hbm):
            @pl.loop(0, TCHUNK // BCH)
            def _(b):
                ch0 = sid * TCHUNK + b * BCH
                pltpu.sync_copy(src_hbm.at[pl.ds(ch0, BCH)], sbuf)
                pltpu.sync_copy(dst_hbm.at[pl.ds(ch0, BCH)], dbuf)

                @pl.loop(0, BCH, step=4)
                def _(j):
                    gds = [
                        pltpu.async_copy(
                            g_hbm.at[sbuf.at[j + q]], rows.at[q], gsem)
                        for q in range(4)
                    ]
                    sds = []
                    for q in range(4):
                        gds[q].wait()
                        sds.append(pltpu.async_copy(
                            rows.at[q], acc.at[dbuf.at[j + q]], ssem,
                            add=True))
                    for d in sds:
                        d.wait()

        @pl.when(cid == 0)
        def _():
            edge_pass(gl_hbm)

        @pl.when(cid == 1)
        def _():
            edge_pass(gr_hbm)

        plsc.subcore_barrier()

        @pl.loop(0, 32)
        def _(kk):
            half = WCHUNK // 2
            pltpu.sync_copy(acc.at[pl.ds(row0 + kk * half, half)], wb)
            pltpu.sync_copy(wb, agg_hbm.at[cid, pl.ds(row0 + kk * half, half)])

    return k(gl, gr, srcp.reshape(NCHUNK, CH), dstp.reshape(NCHUNK, CH), zslab)


def _sc_t(srcp, dstp, dinv):
    @functools.partial(
        pl.kernel,
        out_type=jax.ShapeDtypeStruct((32, NP), jnp.float32),
        mesh=_MESH,
        compiler_params=_CP,
        scratch_types=[
            pltpu.VMEM((NP,), jnp.float32),
            pltpu.VMEM((NP,), jnp.float32),
            pltpu.VMEM((BCH * CH,), jnp.int32),
            pltpu.VMEM((BCH * CH,), jnp.int32),
        ],
    )
    def k(src_hbm, dst_hbm, dinv_hbm, t_hbm, dloc, th, sbuf, dbuf):
        cid = lax.axis_index("c")
        sid = lax.axis_index("s")
        wid = cid * 16 + sid
        zeros = jnp.zeros((16,), jnp.float32)

        pltpu.sync_copy(dinv_hbm, dloc)

        @pl.loop(0, NP, step=16)
        def _(i):
            th[pl.ds(i, 16)] = zeros

        @pl.loop(0, WCHUNK // BCH)
        def _(b):
            e0 = (wid * WCHUNK + b * BCH) * CH
            pltpu.sync_copy(src_hbm.at[pl.ds(e0, BCH * CH)], sbuf)
            pltpu.sync_copy(dst_hbm.at[pl.ds(e0, BCH * CH)], dbuf)

            @pl.loop(0, BCH * CH, step=16)
            def _(i):
                dvals = plsc.load_gather(dloc, [dbuf[pl.ds(i, 16)]])
                plsc.addupdate_scatter(th, [sbuf[pl.ds(i, 16)]], dvals)

        pltpu.sync_copy(th, t_hbm.at[wid])

    return k(srcp, dstp, dinv)


def _tc_matmul(xp, W1):
    def body(x_ref, w_ref, h_ref):
        h_ref[...] = lax.dot_general(
            x_ref[...], w_ref[...], (((1,), (1,)), ((), ())),
            preferred_element_type=jnp.float32)

    return pl.pallas_call(
        body,
        grid=(NBLK,),
        in_specs=[
            pl.BlockSpec((RBT, F), lambda i: (i, 0)),
            pl.BlockSpec((F, F), lambda i: (0, 0)),
        ],
        out_specs=pl.BlockSpec((RBT, F), lambda i: (i, 0)),
        out_shape=jax.ShapeDtypeStruct((NP, F), jnp.float32),
    )(xp, W1)


def _tc_scale(deg32, h):
    def body(deg_ref, h_ref, dinv_ref, dinv1_ref, gl_ref, gr_ref):
        i = pl.program_id(0)
        deg = jnp.sum(deg_ref[...], axis=0) + 1.0
        dv = lax.rsqrt(deg)
        dvc = dv[:, None]
        dinv_ref[...] = dvc
        dinv1_ref[pl.ds(i * RBT, RBT)] = dv
        g = h_ref[...] * dvc
        gl_ref[...] = g[:, :FH]
        gr_ref[...] = g[:, FH:]

    return pl.pallas_call(
        body,
        grid=(NBLK,),
        in_specs=[
            pl.BlockSpec((32, RBT), lambda i: (0, i)),
            pl.BlockSpec((RBT, F), lambda i: (i, 0)),
        ],
        out_specs=[
            pl.BlockSpec((RBT, 1), lambda i: (i, 0)),
            pl.BlockSpec((NP,), lambda i: (0,)),
            pl.BlockSpec((RBT, FH), lambda i: (i, 0)),
            pl.BlockSpec((RBT, FH), lambda i: (i, 0)),
        ],
        out_shape=[
            jax.ShapeDtypeStruct((NP, 1), jnp.float32),
            jax.ShapeDtypeStruct((NP,), jnp.float32),
            jax.ShapeDtypeStruct((NP, FH), jnp.float32),
            jax.ShapeDtypeStruct((NP, FH), jnp.float32),
        ],
    )(deg32, h)


def _tc_final(agg2, gl, gr, dinv, t32, b1, W2T, b2):
    def body(agg_ref, gl_ref, gr_ref, dinv_ref, t_ref, b1_ref,
             w2t_ref, b2_ref, res_ref, acc):
        i = pl.program_id(0)
        dvc = dinv_ref[...]
        pre_l = (agg_ref[0] + gl_ref[...]) * dvc
        pre_r = (agg_ref[1] + gr_ref[...]) * dvc
        pre = jnp.concatenate([pre_l, pre_r], axis=1) + b1_ref[...]
        r1 = jnp.maximum(pre, 0.0)
        tsum = jnp.sum(t_ref[...], axis=0)[:, None]
        w = dvc * (tsum + dvc)
        rowid = lax.broadcasted_iota(jnp.int32, (RBT, 1), 0) + i * RBT
        w = jnp.where(rowid < N, w, 0.0)
        part = jnp.sum(r1 * w, axis=0, keepdims=True)

        @pl.when(i == 0)
        def _():
            acc[...] = part

        @pl.when(i > 0)
        def _():
            acc[...] += part

        @pl.when(i == NBLK - 1)
        def _():
            r = acc[...] * (1.0 / N)
            res_ref[...] = jnp.dot(
                r, w2t_ref[...], preferred_element_type=jnp.float32) + b2_ref[...]

    return pl.pallas_call(
        body,
        grid=(NBLK,),
        in_specs=[
            pl.BlockSpec((2, RBT, FH), lambda i: (0, i, 0)),
            pl.BlockSpec((RBT, FH), lambda i: (i, 0)),
            pl.BlockSpec((RBT, FH), lambda i: (i, 0)),
            pl.BlockSpec((RBT, 1), lambda i: (i, 0)),
            pl.BlockSpec((32, RBT), lambda i: (0, i)),
            pl.BlockSpec((1, F), lambda i: (0, 0)),
            pl.BlockSpec((F, OUT), lambda i: (0, 0)),
            pl.BlockSpec((1, OUT), lambda i: (0, 0)),
        ],
        out_specs=pl.BlockSpec((1, OUT), lambda i: (0, 0)),
        out_shape=jax.ShapeDtypeStruct((1, OUT), jnp.float32),
        scratch_shapes=[pltpu.VMEM((1, F), jnp.float32)],
    )(agg2, gl, gr, dinv, t32, b1, W2T, b2)


def kernel(x, edge_index, W1, b1, W2, b2):
    x = x.astype(jnp.float32)
    xp = jnp.concatenate(
        [x, jnp.zeros((NP - N, F), jnp.float32)], axis=0)

    src = edge_index[0].astype(jnp.int32)
    dst = edge_index[1].astype(jnp.int32)
    padlen = EP - E
    pad_vals = N + (jnp.arange(padlen, dtype=jnp.int32) % (NP - N))
    srcp = jnp.concatenate([src, pad_vals])
    dstp = jnp.concatenate([dst, pad_vals])

    zslab = jnp.zeros((SL, FH), jnp.float32)

    deg32 = _sc_degree(dstp)
    h = _tc_matmul(xp, W1.astype(jnp.float32))
    dinv, dinv1, gl, gr = _tc_scale(deg32, h)
    agg2 = _sc_aggregate(gl, gr, srcp, dstp, zslab)
    t32 = _sc_t(srcp, dstp, dinv1)
    res = _tc_final(
        agg2, gl, gr, dinv, t32,
        b1.astype(jnp.float32).reshape(1, F),
        W2.astype(jnp.float32).T,
        b2.astype(jnp.float32).reshape(1, OUT),
    )
    return res.reshape(OUT)


# speedup vs baseline: 1.1147x; 1.0021x over previous
import functools

import jax
import jax.numpy as jnp
from jax import lax
from jax.experimental import pallas as pl
from jax.experimental.pallas import tpu as pltpu
from jax.experimental.pallas import tpu_sc as plsc

N = 50000
E = 800000
F = 64
FH = 32
OUT = 6

NP = 50176
SL = NP // 16
CH = 128
NCHUNK = 6272
EP = NCHUNK * CH
WCHUNK = NCHUNK // 32
TCHUNK = NCHUNK // 16
BCH = 28
RBT = 1792
NBLK = NP // RBT

_MESH = plsc.VectorSubcoreMesh(core_axis_name="c", subcore_axis_name="s")
_CP = pltpu.CompilerParams(
    needs_layout_passes=False, use_tc_tiling_on_sc=False)


def _sc_degree(dstp):
    @functools.partial(
        pl.kernel,
        out_type=jax.ShapeDtypeStruct((32, NP), jnp.float32),
        mesh=_MESH,
        compiler_params=_CP,
        scratch_types=[
            pltpu.VMEM((NP,), jnp.float32),
            pltpu.VMEM((WCHUNK * CH,), jnp.int32),
        ],
    )
    def k(dst_hbm, deg_hbm, hist, ibuf):
        cid = lax.axis_index("c")
        sid = lax.axis_index("s")
        wid = cid * 16 + sid
        zeros = jnp.zeros((16,), jnp.float32)
        ones = jnp.ones((16,), jnp.float32)

        @pl.loop(0, NP, step=16)
        def _(i):
            hist[pl.ds(i, 16)] = zeros

        pltpu.sync_copy(dst_hbm.at[pl.ds(wid * WCHUNK * CH, WCHUNK * CH)], ibuf)

        @pl.loop(0, WCHUNK * CH, step=16)
        def _(i):
            plsc.addupdate_scatter(hist, [ibuf[pl.ds(i, 16)]], ones)

        pltpu.sync_copy(hist, deg_hbm.at[wid])

    return k(dstp)


def _sc_aggregate(gl, gr, srcp, dstp, zslab):
    @functools.partial(
        pl.kernel,
        out_type=jax.ShapeDtypeStruct((2, NP, FH), jnp.float32),
        mesh=_MESH,
        compiler_params=_CP,
        scratch_types=[
            pltpu.VMEM((BCH, CH), jnp.int32),
            pltpu.VMEM((BCH, CH), jnp.int32),
            pltpu.VMEM((4, CH, FH), jnp.float32),
            pltpu.VMEM((WCHUNK // 2, FH), jnp.float32),
            pltpu.VMEM_SHARED((NP, FH), jnp.float32),
            pltpu.SemaphoreType.DMA,
            pltpu.SemaphoreType.DMA,
        ],
    )
    def k(gl_hbm, gr_hbm, src_hbm, dst_hbm, z_hbm, agg_hbm,
          sbuf, dbuf, rows, wb, acc, gsem, ssem):
        cid = lax.axis_index("c")
        sid = lax.axis_index("s")
        row0 = sid * SL

        pltpu.sync_copy(z_hbm, acc.at[pl.ds(row0, SL)])
        plsc.subcore_barrier()

        def edge_pass(g_hbm):
            @pl.loop(0, TCHUNK // BCH)
            def _(b):
                ch0 = sid * TCHUNK + b * BCH
                pltpu.sync_copy(src_hbm.at[pl.ds(ch0, BCH)], sbuf)
                pltpu.sync_copy(dst_hbm.at[pl.ds(ch0, BCH)], dbuf)

                @pl.loop(0, BCH, step=4)
                def _(j):
                    gds = [
                        pltpu.async_copy(
                            g_hbm.at[sbuf.at[j + q]], rows.at[q], gsem)
                        for q in range(4)
                    ]
                    sds = []
                    for q in range(4):
                        gds[q].wait()
                        sds.append(pltpu.async_copy(
                            rows.at[q], acc.at[dbuf.at[j + q]], ssem,
                            add=True))
                    for d in sds:
                        d.wait()

        @pl.when(cid == 0)
        def _():
            edge_pass(gl_hbm)

        @pl.when(cid == 1)
        def _():
            edge_pass(gr_hbm)

        plsc.subcore_barrier()

        @pl.loop(0, 32)
        def _(kk):
            half = WCHUNK // 2
            pltpu.sync_copy(acc.at[pl.ds(row0 + kk * half, half)], wb)
            pltpu.sync_copy(wb, agg_hbm.at[cid, pl.ds(row0 + kk * half, half)])

    return k(gl, gr, srcp.reshape(NCHUNK, CH), dstp.reshape(NCHUNK, CH), zslab)


def _sc_t(srcp, dstp, dinv):
    @functools.partial(
        pl.kernel,
        out_type=jax.ShapeDtypeStruct((32, NP), jnp.float32),
        mesh=_MESH,
        compiler_params=_CP,
        scratch_types=[
            pltpu.VMEM((NP,), jnp.float32),
            pltpu.VMEM((NP,), jnp.float32),
            pltpu.VMEM((BCH * CH,), jnp.int32),
            pltpu.VMEM((BCH * CH,), jnp.int32),
        ],
    )
    def k(src_hbm, dst_hbm, dinv_hbm, t_hbm, dloc, th, sbuf, dbuf):
        cid = lax.axis_index("c")
        sid = lax.axis_index("s")
        wid = cid * 16 + sid
        zeros = jnp.zeros((16,), jnp.float32)

        pltpu.sync_copy(dinv_hbm, dloc)

        @pl.loop(0, NP, step=16)
        def _(i):
            th[pl.ds(i, 16)] = zeros

        @pl.loop(0, WCHUNK // BCH)
        def _(b):
            e0 = (wid * WCHUNK + b * BCH) * CH
            pltpu.sync_copy(src_hbm.at[pl.ds(e0, BCH * CH)], sbuf)
            pltpu.sync_copy(dst_hbm.at[pl.ds(e0, BCH * CH)], dbuf)

            @pl.loop(0, BCH * CH, step=16)
            def _(i):
                dvals = plsc.load_gather(dloc, [dbuf[pl.ds(i, 16)]])
                plsc.addupdate_scatter(th, [sbuf[pl.ds(i, 16)]], dvals)

        pltpu.sync_copy(th, t_hbm.at[wid])

    return k(srcp, dstp, dinv)


def _tc_matmul(xp, W1):
    def body(x_ref, w_ref, h_ref):
        h_ref[...] = lax.dot_general(
            x_ref[...], w_ref[...], (((1,), (1,)), ((), ())),
            preferred_element_type=jnp.float32)

    return pl.pallas_call(
        body,
        grid=(NBLK,),
        in_specs=[
            pl.BlockSpec((RBT, F), lambda i: (i, 0)),
            pl.BlockSpec((F, F), lambda i: (0, 0)),
        ],
        out_specs=pl.BlockSpec((RBT, F), lambda i: (i, 0)),
        out_shape=jax.ShapeDtypeStruct((NP, F), jnp.float32),
    )(xp, W1)


def _tc_scale(deg32, h):
    def body(deg_ref, h_ref, dinv_ref, dinv1_ref, gl_ref, gr_ref):
        i = pl.program_id(0)
        deg = jnp.sum(deg_ref[...], axis=0) + 1.0
        dv = lax.rsqrt(deg)
        dvc = dv[:, None]
        dinv_ref[...] = dvc
        dinv1_ref[pl.ds(i * RBT, RBT)] = dv
        rowid = lax.broadcasted_iota(jnp.int32, (RBT, 1), 0) + i * RBT
        g = jnp.where(rowid < N, h_ref[...] * dvc, 0.0)
        gl_ref[...] = g[:, :FH]
        gr_ref[...] = g[:, FH:]

    return pl.pallas_call(
        body,
        grid=(NBLK,),
        in_specs=[
            pl.BlockSpec((32, RBT), lambda i: (0, i)),
            pl.BlockSpec((RBT, F), lambda i: (i, 0)),
        ],
        out_specs=[
            pl.BlockSpec((RBT, 1), lambda i: (i, 0)),
            pl.BlockSpec((NP,), lambda i: (0,)),
            pl.BlockSpec((RBT, FH), lambda i: (i, 0)),
            pl.BlockSpec((RBT, FH), lambda i: (i, 0)),
        ],
        out_shape=[
            jax.ShapeDtypeStruct((NP, 1), jnp.float32),
            jax.ShapeDtypeStruct((NP,), jnp.float32),
            jax.ShapeDtypeStruct((NP, FH), jnp.float32),
            jax.ShapeDtypeStruct((NP, FH), jnp.float32),
        ],
    )(deg32, h)


def _tc_final(agg2, gl, gr, dinv, t32, b1, W2T, b2):
    def body(agg_ref, gl_ref, gr_ref, dinv_ref, t_ref, b1_ref,
             w2t_ref, b2_ref, res_ref, acc):
        i = pl.program_id(0)
        dvc = dinv_ref[...]
        pre_l = (agg_ref[0] + gl_ref[...]) * dvc
        pre_r = (agg_ref[1] + gr_ref[...]) * dvc
        pre = jnp.concatenate([pre_l, pre_r], axis=1) + b1_ref[...]
        r1 = jnp.maximum(pre, 0.0)
        tsum = jnp.sum(t_ref[...], axis=0)[:, None]
        w = dvc * (tsum + dvc)
        rowid = lax.broadcasted_iota(jnp.int32, (RBT, 1), 0) + i * RBT
        w = jnp.where(rowid < N, w, 0.0)
        part = jnp.sum(r1 * w, axis=0, keepdims=True)

        @pl.when(i == 0)
        def _():
            acc[...] = part

        @pl.when(i > 0)
        def _():
            acc[...] += part

        @pl.when(i == NBLK - 1)
        def _():
            r = acc[...] * (1.0 / N)
            res_ref[...] = jnp.dot(
                r, w2t_ref[...], preferred_element_type=jnp.float32) + b2_ref[...]

    return pl.pallas_call(
        body,
        grid=(NBLK,),
        in_specs=[
            pl.BlockSpec((2, RBT, FH), lambda i: (0, i, 0)),
            pl.BlockSpec((RBT, FH), lambda i: (i, 0)),
            pl.BlockSpec((RBT, FH), lambda i: (i, 0)),
            pl.BlockSpec((RBT, 1), lambda i: (i, 0)),
            pl.BlockSpec((32, RBT), lambda i: (0, i)),
            pl.BlockSpec((1, F), lambda i: (0, 0)),
            pl.BlockSpec((F, OUT), lambda i: (0, 0)),
            pl.BlockSpec((1, OUT), lambda i: (0, 0)),
        ],
        out_specs=pl.BlockSpec((1, OUT), lambda i: (0, 0)),
        out_shape=jax.ShapeDtypeStruct((1, OUT), jnp.float32),
        scratch_shapes=[pltpu.VMEM((1, F), jnp.float32)],
    )(agg2, gl, gr, dinv, t32, b1, W2T, b2)


def kernel(x, edge_index, W1, b1, W2, b2):
    xp = x.astype(jnp.float32)

    src = edge_index[0].astype(jnp.int32)
    dst = edge_index[1].astype(jnp.int32)
    padlen = EP - E
    pad_vals = N + (jnp.arange(padlen, dtype=jnp.int32) % (NP - N))
    srcp = jnp.concatenate([src, pad_vals])
    dstp = jnp.concatenate([dst, pad_vals])

    zslab = jnp.zeros((SL, FH), jnp.float32)

    deg32 = _sc_degree(dstp)
    h = _tc_matmul(xp, W1.astype(jnp.float32))
    dinv, dinv1, gl, gr = _tc_scale(deg32, h)
    agg2 = _sc_aggregate(gl, gr, srcp, dstp, zslab)
    t32 = _sc_t(srcp, dstp, dinv1)
    res = _tc_final(
        agg2, gl, gr, dinv, t32,
        b1.astype(jnp.float32).reshape(1, F),
        W2.astype(jnp.float32).T,
        b2.astype(jnp.float32).reshape(1, OUT),
    )
    return res.reshape(OUT)
